# baseline (speedup 1.0000x reference)
import functools
import math

import numpy as np
import jax
import jax.numpy as jnp
from jax import lax
from jax.experimental import pallas as pl
from jax.experimental.pallas import tpu as pltpu
from jax.experimental.pallas import tpu_sc as plsc

_LANES = 16


def _positional_encoding_np(seq_len, d_model):
    pos = np.arange(seq_len, dtype=np.float32)[:, None]
    i = np.arange(0, d_model, 2, dtype=np.float32)[None, :]
    angles = pos / np.power(10000.0, i / d_model)
    pe = np.zeros((seq_len, d_model), dtype=np.float32)
    pe[:, 0::2] = np.sin(angles)
    pe[:, 1::2] = np.cos(angles)
    return pe


@functools.lru_cache(maxsize=None)
def _build(B, L, D, V):
    info = plsc.get_sparse_core_info()
    NC, NS = info.num_cores, info.num_subcores
    NW = NC * NS
    total = B * L
    assert total % NW == 0
    per_w = total // NW
    assert per_w % L == 0, "each worker range must start on a sequence boundary"
    reps = 2
    ch = reps * L
    assert per_w % ch == 0
    n_chunks = per_w // ch
    assert D % _LANES == 0
    dblk = D // _LANES

    mesh = plsc.VectorSubcoreMesh(core_axis_name="c", subcore_axis_name="s")

    @functools.partial(
        pl.kernel,
        mesh=mesh,
        compiler_params=pltpu.CompilerParams(use_tc_tiling_on_sc=False),
        out_type=jax.ShapeDtypeStruct((total, D), jnp.float32),
        scratch_types=[
            pltpu.VMEM((ch,), jnp.int32),
            pltpu.VMEM((ch, D), jnp.float32),
            pltpu.VMEM((L, D), jnp.float32),
            pltpu.SemaphoreType.DMA,
        ],
    )
    def _k(x_hbm, pe_hbm, table_hbm, out_hbm, idx_v, rows_v, pe_v, sem):
        wid = lax.axis_index("s") * NC + lax.axis_index("c")
        base = wid * per_w
        pltpu.sync_copy(pe_hbm, pe_v)

        def chunk_body(i, _):
            off = base + i * ch
            pltpu.sync_copy(x_hbm.at[pl.ds(off, ch)], idx_v)
            pltpu.async_copy(table_hbm.at[idx_v], rows_v, sem).wait()

            def add_body(l, _):
                for rep in range(reps):
                    for j in range(dblk):
                        s = pl.ds(j * _LANES, _LANES)
                        rows_v[rep * L + l, s] = rows_v[rep * L + l, s] + pe_v[l, s]
                return 0

            lax.fori_loop(0, L, add_body, 0)
            pltpu.sync_copy(rows_v, out_hbm.at[pl.ds(off, ch)])
            return 0

        lax.fori_loop(0, n_chunks, chunk_body, 0)

    return _k


def kernel(x, table):
    B, L = x.shape
    V, D = table.shape
    pe = jnp.asarray(_positional_encoding_np(L, D))
    xf = x.reshape(-1).astype(jnp.int32)
    out = _build(B, L, D, V)(xf, pe, table)
    return out.reshape(B, L, D)


# speedup vs baseline: 1.0848x; 1.0848x over previous
import functools

import numpy as np
import jax
import jax.numpy as jnp
from jax import lax
from jax.experimental import pallas as pl
from jax.experimental.pallas import tpu as pltpu
from jax.experimental.pallas import tpu_sc as plsc

_LANES = 16


def _positional_encoding_np(seq_len, d_model):
    pos = np.arange(seq_len, dtype=np.float32)[:, None]
    i = np.arange(0, d_model, 2, dtype=np.float32)[None, :]
    angles = pos / np.power(10000.0, i / d_model)
    pe = np.zeros((seq_len, d_model), dtype=np.float32)
    pe[:, 0::2] = np.sin(angles)
    pe[:, 1::2] = np.cos(angles)
    return pe


@functools.lru_cache(maxsize=None)
def _build(B, L, D, V):
    info = plsc.get_sparse_core_info()
    NC, NS = info.num_cores, info.num_subcores
    NW = NC * NS
    total = B * L
    assert total % NW == 0
    per_w = total // NW
    assert per_w % L == 0, "each worker range must start on a sequence boundary"
    reps = 2
    ch = reps * L
    RB = 4
    assert per_w % (ch * RB) == 0
    n_chunks = per_w // ch
    assert D % _LANES == 0
    dblk = D // _LANES

    mesh = plsc.VectorSubcoreMesh(core_axis_name="c", subcore_axis_name="s")

    @functools.partial(
        pl.kernel,
        mesh=mesh,
        compiler_params=pltpu.CompilerParams(use_tc_tiling_on_sc=False),
        out_type=jax.ShapeDtypeStruct((total, D), jnp.float32),
        scratch_types=(
            [pltpu.VMEM((ch,), jnp.int32) for _ in range(RB)]
            + [pltpu.VMEM((ch, D), jnp.float32) for _ in range(RB)]
            + [pltpu.VMEM((L, D), jnp.float32)]
            + [pltpu.SemaphoreType.DMA for _ in range(2 * RB)]
        ),
    )
    def _k(x_hbm, pe_hbm, table_hbm, out_hbm, *scratch):
        idx_v = scratch[:RB]
        rows_v = scratch[RB:2 * RB]
        pe_v = scratch[2 * RB]
        gsem = scratch[2 * RB + 1:2 * RB + 1 + RB]
        osem = scratch[2 * RB + 1 + RB:]

        wid = lax.axis_index("s") * NC + lax.axis_index("c")
        base = wid * per_w
        pltpu.sync_copy(pe_hbm, pe_v)

        for r in range(RB - 1):
            off = base + r * ch
            pltpu.sync_copy(x_hbm.at[pl.ds(off, ch)], idx_v[r])
            pltpu.async_copy(table_hbm.at[idx_v[r]], rows_v[r], gsem[r])

        @pl.loop(0, n_chunks, step=RB)
        def _outer(io):
            for r in range(RB):
                i = io + r
                pre = (r + RB - 1) % RB

                @pl.when(i + RB - 1 < n_chunks)
                def _():
                    @pl.when(i >= 1)
                    def _():
                        pltpu.make_async_copy(
                            rows_v[pre],
                            out_hbm.at[pl.ds(base + (i - 1) * ch, ch)],
                            osem[pre],
                        ).wait()

                    off2 = base + (i + RB - 1) * ch
                    pltpu.sync_copy(x_hbm.at[pl.ds(off2, ch)], idx_v[pre])
                    pltpu.async_copy(table_hbm.at[idx_v[pre]], rows_v[pre],
                                     gsem[pre])

                pltpu.make_async_copy(table_hbm.at[idx_v[r]], rows_v[r],
                                      gsem[r]).wait()

                @pl.loop(0, L)
                def _add(l):
                    for rep in range(reps):
                        for j in range(dblk):
                            s = pl.ds(j * _LANES, _LANES)
                            rows_v[r][rep * L + l, s] = (
                                rows_v[r][rep * L + l, s] + pe_v[l, s])

                pltpu.async_copy(rows_v[r],
                                 out_hbm.at[pl.ds(base + i * ch, ch)], osem[r])

        for r in range(RB):
            i_last = n_chunks - RB + r
            pltpu.make_async_copy(
                rows_v[r], out_hbm.at[pl.ds(base + i_last * ch, ch)],
                osem[r]).wait()

    return _k


def kernel(x, table):
    B, L = x.shape
    V, D = table.shape
    pe = jnp.asarray(_positional_encoding_np(L, D))
    xf = x.reshape(-1).astype(jnp.int32)
    out = _build(B, L, D, V)(xf, pe, table)
    return out.reshape(B, L, D)
